# baseline (speedup 1.0000x reference)
import functools

import jax
import jax.numpy as jnp
from jax import lax
from jax.experimental import pallas as pl
from jax.experimental.pallas import tpu as pltpu


def _build_w_kernel(x_ref, w1_ref, w2_ref, w_ref, *, dim):
    w_ref[:, :dim] = w1_ref[...]
    xw = jnp.dot(x_ref[...].astype(jnp.bfloat16),
                 w2_ref[...].astype(jnp.bfloat16),
                 preferred_element_type=jnp.float32)
    w_ref[:, dim:] = xw


def _stage1_kernel(a_ref, w_ref, whr_ref, y_ref, abf_ref, acc_ref, *,
                   k_steps, k_rem, dim):
    k = pl.program_id(1)

    @pl.when(k == 0)
    def _():
        acc_ref[...] = jnp.zeros_like(acc_ref)

    @pl.when(k < k_steps - 1)
    def _():
        a_bf = a_ref[...].astype(jnp.bfloat16)
        abf_ref[...] = a_bf
        acc_ref[...] += jnp.dot(a_bf, w_ref[...].astype(jnp.bfloat16),
                                preferred_element_type=jnp.float32)

    @pl.when(k == k_steps - 1)
    def _():
        bm, bk = a_ref.shape
        col = lax.broadcasted_iota(jnp.int32, (bm, bk), 1)
        a_bf = jnp.where(col < k_rem, a_ref[...], 0.0).astype(jnp.bfloat16)
        abf_ref[...] = a_bf
        row = lax.broadcasted_iota(jnp.int32, w_ref.shape, 0)
        w_bf = jnp.where(row < k_rem, w_ref[...], 0.0).astype(jnp.bfloat16)
        acc = acc_ref[...] + jnp.dot(a_bf, w_bf,
                                     preferred_element_type=jnp.float32)
        a_act = jax.nn.relu(acc[:, :dim])
        b_act = jax.nn.relu(acc[:, dim:])
        t = jax.nn.sigmoid(
            jnp.sum(b_act * whr_ref[0:1, :], axis=1, keepdims=True))
        y_ref[...] = t * a_act + (1.0 - t) * b_act


def _stage2_kernel(abf_ref, y_ref, out_ref, acc_ref, *, k_steps, k_rem):
    k = pl.program_id(1)

    @pl.when(k == 0)
    def _():
        acc_ref[...] = jnp.zeros_like(acc_ref)

    @pl.when(k < k_steps - 1)
    def _():
        acc_ref[...] += jnp.dot(abf_ref[...],
                                y_ref[...].astype(jnp.bfloat16),
                                preferred_element_type=jnp.float32)

    @pl.when(k == k_steps - 1)
    def _():
        row = lax.broadcasted_iota(jnp.int32, y_ref.shape, 0)
        y = jnp.where(row < k_rem, y_ref[...], 0.0).astype(jnp.bfloat16)
        out_ref[...] = acc_ref[...] + jnp.dot(
            abf_ref[...], y, preferred_element_type=jnp.float32)


def _pick_bm(n, target):
    for b in range(min(target, n), 7, -1):
        if n % b == 0 and b % 8 == 0:
            return b
    return n


def kernel(x, A, w1, w2, wh):
    n, d_in = x.shape
    dim = w1.shape[1]

    bm = 1024
    bk = 2048
    m_steps = -(-n // bm)
    k_steps = -(-n // bk)
    k_rem = n - (k_steps - 1) * bk

    bw = _pick_bm(n, 2000)
    W = pl.pallas_call(
        functools.partial(_build_w_kernel, dim=dim),
        grid=(n // bw,),
        in_specs=[
            pl.BlockSpec((bw, d_in), lambda i: (i, 0)),
            pl.BlockSpec((bw, dim), lambda i: (i, 0)),
            pl.BlockSpec((d_in, dim), lambda i: (0, 0)),
        ],
        out_specs=pl.BlockSpec((bw, 2 * dim), lambda i: (i, 0)),
        out_shape=jax.ShapeDtypeStruct((n, 2 * dim), jnp.float32),
    )(x, w1, w2)

    whr = jnp.broadcast_to(wh.reshape(1, dim), (8, dim))

    y, Abf = pl.pallas_call(
        functools.partial(_stage1_kernel, k_steps=k_steps, k_rem=k_rem,
                          dim=dim),
        grid=(m_steps, k_steps),
        in_specs=[
            pl.BlockSpec((bm, bk), lambda i, k: (i, k)),
            pl.BlockSpec((bk, 2 * dim), lambda i, k: (k, 0)),
            pl.BlockSpec((8, dim), lambda i, k: (0, 0)),
        ],
        out_specs=[
            pl.BlockSpec((bm, dim), lambda i, k: (i, 0)),
            pl.BlockSpec((bm, bk), lambda i, k: (i, k)),
        ],
        out_shape=[
            jax.ShapeDtypeStruct((n, dim), jnp.float32),
            jax.ShapeDtypeStruct((m_steps * bm, k_steps * bk),
                                 jnp.bfloat16),
        ],
        scratch_shapes=[pltpu.VMEM((bm, 2 * dim), jnp.float32)],
        compiler_params=pltpu.CompilerParams(
            dimension_semantics=("parallel", "arbitrary")),
    )(A, W, whr)

    out = pl.pallas_call(
        functools.partial(_stage2_kernel, k_steps=k_steps, k_rem=k_rem),
        grid=(m_steps, k_steps),
        in_specs=[
            pl.BlockSpec((bm, bk), lambda i, k: (i, k)),
            pl.BlockSpec((bk, dim), lambda i, k: (k, 0)),
        ],
        out_specs=pl.BlockSpec((bm, dim), lambda i, k: (i, 0)),
        out_shape=jax.ShapeDtypeStruct((n, dim), jnp.float32),
        scratch_shapes=[pltpu.VMEM((bm, dim), jnp.float32)],
        compiler_params=pltpu.CompilerParams(
            dimension_semantics=("parallel", "arbitrary")),
    )(Abf, y)

    return out
